# baseline (speedup 1.0000x reference)
import functools

import jax
import jax.numpy as jnp
from jax import lax
from jax.experimental import pallas as pl
from jax.experimental.pallas import tpu as pltpu
from jax.experimental.pallas import tpu_sc as plsc

_N = 10000
_E = 320000
_F = 128
_S = 4
_H = 32

_NC = 2
_NS = 16
_NW = _NC * _NS
_CB = 128
_NCH = _E // _CB
_CPW = _NCH // _NW
_DB = 3
_XTRA = _NCH - _CPW * _NW
_AW = _N * 8

_ST = 1000
_NST = _N // _ST
_TT = 2000
_NTT = _N // _TT


def _sc_body(ei3_hbm, ea3_hbm, zero_hbm, out_hbm, idx_v, ea_v, val_v,
             tb_v, tt_v, acc_sh, sem):
    cid = lax.axis_index("c")
    sid = lax.axis_index("s")
    wid = sid * _NC + cid

    @pl.when(sid < _NST)
    def _zero():
        pltpu.sync_copy(zero_hbm.at[pl.ds(sid * _ST, _ST)],
                        acc_sh.at[pl.ds(sid * _ST, _ST)])

    lane = lax.iota(jnp.int32, 16)
    ones16 = jnp.full((16,), 1.0, jnp.float32)
    col4 = jnp.full((16,), 4, jnp.int32)
    for b in range(_DB):
        pltpu.sync_copy(zero_hbm.at[pl.ds(0, _CB)], val_v.at[b])
        for g in range(8):
            plsc.store_scatter(val_v.at[b], [lane + 16 * g, col4], ones16)

    plsc.subcore_barrier()

    pltpu.sync_copy(ei3_hbm.at[pl.ds(wid * _CPW, _CPW), pl.ds(0, 1)],
                    idx_v.at[pl.ds(0, _CPW)])
    pltpu.sync_copy(ea3_hbm.at[pl.ds(wid * _CPW, _CPW)],
                    ea_v.at[pl.ds(0, _CPW)])

    def build(j, b):
        for g in range(8):
            e_idx = lane + 16 * g
            for s in range(_S):
                v = ea_v[j, s, pl.ds(16 * g, 16)]
                plsc.store_scatter(
                    val_v.at[b], [e_idx, jnp.full((16,), s, jnp.int32)], v)

    def do_chunk(j):
        build(j, 0)
        pltpu.sync_copy(val_v.at[0], acc_sh.at[idx_v.at[j, 0]], add=True)

    def group(jj, carry):
        j0 = jj * _DB
        cps = []
        for b in range(_DB):
            build(j0 + b, b)
            cps.append(pltpu.async_copy(
                val_v.at[b], acc_sh.at[idx_v.at[j0 + b, 0]], sem, add=True))
        for cp in cps:
            cp.wait()
        return carry

    lax.fori_loop(0, _CPW // _DB, group, 0)

    @pl.when(wid < _XTRA)
    def _extra():
        pltpu.sync_copy(ei3_hbm.at[pl.ds(_NW * _CPW + wid, 1), pl.ds(0, 1)],
                        idx_v.at[pl.ds(0, 1)])
        pltpu.sync_copy(ea3_hbm.at[pl.ds(_NW * _CPW + wid, 1)],
                        ea_v.at[pl.ds(0, 1)])
        do_chunk(0)

    plsc.subcore_barrier()

    @pl.when(sid < _NTT)
    def _writeout():
        pltpu.sync_copy(acc_sh.at[pl.ds(sid * _TT, _TT)], tb_v)

        def tbody(k, carry):
            n_idx = lane + k * 16
            for col in range(8):
                v = plsc.load_gather(tb_v, [n_idx, jnp.full((16,), col,
                                                            jnp.int32)])
                tt_v[col, pl.ds(k * 16, 16)] = v
            return carry

        lax.fori_loop(0, _TT // 16, tbody, 0)
        pltpu.sync_copy(tt_v, out_hbm.at[cid, :, pl.ds(sid * _TT, _TT)])


def _make_sc_kernel():
    mesh = plsc.VectorSubcoreMesh(core_axis_name="c", subcore_axis_name="s")
    return functools.partial(
        pl.kernel,
        mesh=mesh,
        compiler_params=pltpu.CompilerParams(use_tc_tiling_on_sc=False,
                                             needs_layout_passes=False),
        out_type=jax.ShapeDtypeStruct((_NC, 8, _N), jnp.float32),
        scratch_types=[
            pltpu.VMEM((_CPW + 1, 1, _CB), jnp.int32),
            pltpu.VMEM((_CPW + 1, _S, _CB), jnp.float32),
            pltpu.VMEM((_DB, _CB, 8), jnp.float32),
            pltpu.VMEM((_TT, 8), jnp.float32),
            pltpu.VMEM((8, _TT), jnp.float32),
            pltpu.VMEM_SHARED((_N, 8), jnp.float32),
            pltpu.SemaphoreType.DMA,
        ],
    )(_sc_body)


def _tc_body(ct_hbm, x_ref, m2_ref, cb_ref, dw_ref, db_ref, out_ref,
             ct_ref, sem):
    pltpu.make_async_copy(ct_hbm, ct_ref, sem).start()
    pltpu.make_async_copy(ct_hbm, ct_ref, sem).wait()
    c = ct_ref[0] + ct_ref[1]
    row = lax.broadcasted_iota(jnp.int32, c.shape, 0)
    c = jnp.where(row == 5, 1.0, c)
    a8 = lax.dot_general(
        c, x_ref[...], (((1,), (0,)), ((), ())),
        preferred_element_type=jnp.float32)
    pooled = jnp.float32(_N) * cb_ref[0:1, :]
    for s in range(6):
        pooled = pooled + jnp.dot(
            a8[s:s + 1, :], m2_ref[s * _F:(s + 1) * _F, :],
            preferred_element_type=jnp.float32)
    y = jnp.dot(pooled, dw_ref[...],
                preferred_element_type=jnp.float32) + db_ref[0:1, :]
    out_ref[...] = jnp.broadcast_to(y, out_ref.shape)


def kernel(x, edge_index, edge_attr, K, b_k, root_kernel, conv_bias,
           dense_w, dense_b):
    ei3 = edge_index.reshape(2, _NCH, _CB).transpose(1, 0, 2)
    ea3 = edge_attr.T.reshape(_S, _NCH, _CB).transpose(1, 0, 2)
    zeros_n8 = jnp.zeros((_N, 8), jnp.float32)

    ct = _make_sc_kernel()(ei3, ea3, zeros_n8)

    k3 = K.reshape(_S, _F, _H)
    m = jnp.concatenate(
        [k3, b_k.reshape(1, _F, _H), root_kernel[None],
         jnp.zeros((2, _F, _H), jnp.float32)], axis=0)
    m2 = m.reshape(8 * _F, _H)
    cb8 = jnp.zeros((8, _H), jnp.float32).at[0].set(conv_bias)
    dwp = jnp.zeros((_H, _F), jnp.float32).at[:, :3].set(dense_w)
    dbp = jnp.zeros((8, _F), jnp.float32).at[0, :3].set(dense_b)

    out = pl.pallas_call(
        _tc_body,
        in_specs=[
            pl.BlockSpec(memory_space=pl.ANY),
            pl.BlockSpec(memory_space=pltpu.VMEM),
            pl.BlockSpec(memory_space=pltpu.VMEM),
            pl.BlockSpec(memory_space=pltpu.VMEM),
            pl.BlockSpec(memory_space=pltpu.VMEM),
            pl.BlockSpec(memory_space=pltpu.VMEM),
        ],
        scratch_shapes=[pltpu.VMEM((_NC, 8, _N), jnp.float32),
                        pltpu.SemaphoreType.DMA],
        out_shape=jax.ShapeDtypeStruct((8, _F), jnp.float32),
    )(ct, x, m2, cb8, dwp, dbp)
    return out[0, :3]


# speedup vs baseline: 1.0213x; 1.0213x over previous
import functools

import jax
import jax.numpy as jnp
from jax import lax
from jax.experimental import pallas as pl
from jax.experimental.pallas import tpu as pltpu
from jax.experimental.pallas import tpu_sc as plsc

_N = 10000
_E = 320000
_F = 128
_S = 4
_H = 32

_NC = 2
_NS = 16
_NW = _NC * _NS
_CB = 128
_NCH = _E // _CB
_CPW = _NCH // _NW
_DB = 3
_HPW = 39
_XTRA = _NCH - _CPW * _NW
_AW = _N * 8

_ST = 1000
_NST = _N // _ST
_TT = 2000
_NTT = _N // _TT


def _sc_body(ei3_hbm, ea3_hbm, zero_hbm, out_hbm, idx_v, ea_v, val_v,
             tb_v, tt_v, acc_sh, sem, sem2):
    cid = lax.axis_index("c")
    sid = lax.axis_index("s")
    wid = sid * _NC + cid

    @pl.when(sid < _NST)
    def _zero():
        pltpu.sync_copy(zero_hbm.at[pl.ds(sid * _ST, _ST)],
                        acc_sh.at[pl.ds(sid * _ST, _ST)])

    lane = lax.iota(jnp.int32, 16)
    ones16 = jnp.full((16,), 1.0, jnp.float32)
    col4 = jnp.full((16,), 4, jnp.int32)
    for b in range(_DB):
        pltpu.sync_copy(zero_hbm.at[pl.ds(0, _CB)], val_v.at[b])
        for g in range(8):
            plsc.store_scatter(val_v.at[b], [lane + 16 * g, col4], ones16)

    plsc.subcore_barrier()

    pltpu.sync_copy(ei3_hbm.at[pl.ds(wid * _CPW, _CPW), pl.ds(0, 1)],
                    idx_v.at[pl.ds(0, _CPW)])
    pltpu.sync_copy(ea3_hbm.at[pl.ds(wid * _CPW, _HPW)],
                    ea_v.at[pl.ds(0, _HPW)])
    cp_stage = pltpu.async_copy(
        ea3_hbm.at[pl.ds(wid * _CPW + _HPW, _CPW - _HPW)],
        ea_v.at[pl.ds(_HPW, _CPW - _HPW)], sem2)

    def build(j, b):
        for g in range(8):
            e_idx = lane + 16 * g
            for s in range(_S):
                v = ea_v[j, s, pl.ds(16 * g, 16)]
                plsc.store_scatter(
                    val_v.at[b], [e_idx, jnp.full((16,), s, jnp.int32)], v)

    def do_chunk(j):
        build(j, 0)
        pltpu.sync_copy(val_v.at[0], acc_sh.at[idx_v.at[j, 0]], add=True)

    def group(jj, carry):
        j0 = jj * _DB
        cps = []
        for b in range(_DB):
            build(j0 + b, b)
            cps.append(pltpu.async_copy(
                val_v.at[b], acc_sh.at[idx_v.at[j0 + b, 0]], sem, add=True))
        for cp in cps:
            cp.wait()
        return carry

    lax.fori_loop(0, _HPW // _DB, group, 0)
    cp_stage.wait()
    lax.fori_loop(_HPW // _DB, _CPW // _DB, group, 0)

    @pl.when(wid < _XTRA)
    def _extra():
        pltpu.sync_copy(ei3_hbm.at[pl.ds(_NW * _CPW + wid, 1), pl.ds(0, 1)],
                        idx_v.at[pl.ds(0, 1)])
        pltpu.sync_copy(ea3_hbm.at[pl.ds(_NW * _CPW + wid, 1)],
                        ea_v.at[pl.ds(0, 1)])
        do_chunk(0)

    plsc.subcore_barrier()

    @pl.when(sid < _NTT)
    def _writeout():
        pltpu.sync_copy(acc_sh.at[pl.ds(sid * _TT, _TT)], tb_v)

        def tbody(k, carry):
            n_idx = lane + k * 16
            for col in range(8):
                v = plsc.load_gather(tb_v, [n_idx, jnp.full((16,), col,
                                                            jnp.int32)])
                tt_v[col, pl.ds(k * 16, 16)] = v
            return carry

        lax.fori_loop(0, _TT // 16, tbody, 0)
        pltpu.sync_copy(tt_v, out_hbm.at[cid, :, pl.ds(sid * _TT, _TT)])


def _make_sc_kernel():
    mesh = plsc.VectorSubcoreMesh(core_axis_name="c", subcore_axis_name="s")
    return functools.partial(
        pl.kernel,
        mesh=mesh,
        compiler_params=pltpu.CompilerParams(use_tc_tiling_on_sc=False,
                                             needs_layout_passes=False),
        out_type=jax.ShapeDtypeStruct((_NC, 8, _N), jnp.float32),
        scratch_types=[
            pltpu.VMEM((_CPW + 1, 1, _CB), jnp.int32),
            pltpu.VMEM((_CPW + 1, _S, _CB), jnp.float32),
            pltpu.VMEM((_DB, _CB, 8), jnp.float32),
            pltpu.VMEM((_TT, 8), jnp.float32),
            pltpu.VMEM((8, _TT), jnp.float32),
            pltpu.VMEM_SHARED((_N, 8), jnp.float32),
            pltpu.SemaphoreType.DMA,
            pltpu.SemaphoreType.DMA,
        ],
    )(_sc_body)


def _tc_body(ct_ref, x_ref, m2_ref, cb_ref, dw_ref, db_ref, out_ref):
    c = ct_ref[0] + ct_ref[1]
    row = lax.broadcasted_iota(jnp.int32, c.shape, 0)
    c = jnp.where(row == 5, 1.0, c)
    a8 = lax.dot_general(
        c, x_ref[...], (((1,), (0,)), ((), ())),
        preferred_element_type=jnp.float32)
    pooled = jnp.float32(_N) * cb_ref[0:1, :]
    for s in range(6):
        pooled = pooled + jnp.dot(
            a8[s:s + 1, :], m2_ref[s * _F:(s + 1) * _F, :],
            preferred_element_type=jnp.float32)
    y = jnp.dot(pooled, dw_ref[...],
                preferred_element_type=jnp.float32) + db_ref[0:1, :]
    out_ref[...] = jnp.broadcast_to(y, out_ref.shape)


def kernel(x, edge_index, edge_attr, K, b_k, root_kernel, conv_bias,
           dense_w, dense_b):
    ei3 = edge_index.reshape(2, _NCH, _CB).transpose(1, 0, 2)
    ea3 = edge_attr.T.reshape(_S, _NCH, _CB).transpose(1, 0, 2)
    zeros_n8 = jnp.zeros((_N, 8), jnp.float32)

    ct = _make_sc_kernel()(ei3, ea3, zeros_n8)

    k3 = K.reshape(_S, _F, _H)
    m = jnp.concatenate(
        [k3, b_k.reshape(1, _F, _H), root_kernel[None],
         jnp.zeros((2, _F, _H), jnp.float32)], axis=0)
    m2 = m.reshape(8 * _F, _H)
    cb8 = jnp.zeros((8, _H), jnp.float32).at[0].set(conv_bias)
    dwp = jnp.zeros((_H, _F), jnp.float32).at[:, :3].set(dense_w)
    dbp = jnp.zeros((8, _F), jnp.float32).at[0, :3].set(dense_b)

    out = pl.pallas_call(
        _tc_body,
        out_shape=jax.ShapeDtypeStruct((8, _F), jnp.float32),
    )(ct, x, m2, cb8, dwp, dbp)
    return out[0, :3]
